# baseline (speedup 1.0000x reference)
import functools

import jax
import jax.numpy as jnp
from jax import lax
from jax.experimental import pallas as pl
from jax.experimental.pallas import tpu as pltpu
from jax.experimental.pallas import tpu_sc as plsc

D_MODEL = 64
LANES = 16
SCALE = 8.0

VOCAB_TILES_FULL = 7812
VOCAB_TAIL = 999936
TILES_PER_W = 245


def _iota16():
    return lax.iota(jnp.int32, LANES)


def _splat16(v):
    return jnp.zeros((LANES,), jnp.int32) + v


def _transpose_body(tt_hbm, scr_hbm, slab_v, trows_v, sem, *, nc):
    wid = lax.axis_index("s") * nc + lax.axis_index("c")
    t0 = wid * TILES_PER_W
    t1 = jnp.minimum(t0 + TILES_PER_W, VOCAB_TILES_FULL)
    iota = _iota16()

    def do_slab(i0, nrows):
        def il_body(il, c2):
            ilv = _splat16(il)
            for g in range(D_MODEL // LANES):
                v = plsc.load_gather(slab_v, [g * LANES + iota, ilv])
                trows_v[pl.ds(il * 128 + g * LANES, LANES)] = v * SCALE
            return c2

        lax.fori_loop(0, nrows, il_body, 0, unroll=4)

    def col_body(c, carry):
        i0 = c * 128
        pltpu.sync_copy(tt_hbm.at[:, pl.ds(i0, 128)], slab_v)
        do_slab(i0, 128)
        pltpu.sync_copy(trows_v, scr_hbm.at[pl.ds(i0 * 128, 128 * 128)])
        return carry

    lax.fori_loop(t0, t1, col_body, 0)

    @pl.when(wid == 31)
    def _tail():
        t_start = pl.multiple_of(wid * 0 + VOCAB_TAIL, 128)
        pltpu.sync_copy(tt_hbm.at[:, pl.ds(t_start, 128)], slab_v)
        do_slab(VOCAB_TAIL, 64)
        pltpu.sync_copy(
            trows_v.at[pl.ds(0, 64 * 128)],
            scr_hbm.at[pl.ds(VOCAB_TAIL * 128, 64 * 128)],
        )


def _gather_body(scr_hbm, idx_hbm, out_hbm, idx_v, rows_v, trows_v, sem, *, nc):
    wid = lax.axis_index("s") * nc + lax.axis_index("c")
    bh0 = wid * 4
    iota = _iota16()

    def task_body(t, carry):
        s = t // 2
        p = t % 2
        b0 = (bh0 + 2 * p) * 128
        pltpu.sync_copy(idx_hbm.at[pl.ds(s * 16384 + b0, 256)], idx_v)
        pltpu.async_copy(scr_hbm.at[idx_v], rows_v, sem).wait()

        def bg_body(bg, c2):
            bvec = bg * LANES + iota
            bh_ = bg // 8
            blo0 = (bg % 8) * LANES
            for d in range(D_MODEL):
                v = plsc.load_gather(rows_v, [bvec, _splat16(d)])
                off = (d // 8) * 2048 + bh_ * 1024 + (d % 8) * 128 + blo0
                trows_v[pl.ds(off, LANES)] = v
            return c2

        lax.fori_loop(0, 16, bg_body, 0)

        obase = s * 1048576 + (bh0 + 2 * p) * 1024
        for dh in range(8):
            pltpu.sync_copy(
                trows_v.at[pl.ds(dh * 2048, 2048)],
                out_hbm.at[pl.ds(obase + dh * 131072, 2048)],
            )
        return carry

    lax.fori_loop(0, 100, task_body, 0)


def kernel(x, table):
    b, s = x.shape
    n = b * s
    idx_t = jnp.transpose(x).reshape(n).astype(jnp.int32)
    tt = jnp.transpose(table)

    info = plsc.get_sparse_core_info()
    nc = info.num_cores
    mesh = plsc.VectorSubcoreMesh(core_axis_name="c", subcore_axis_name="s")
    params = pltpu.CompilerParams(
        use_tc_tiling_on_sc=True,
        disable_bounds_checks=True,
        needs_layout_passes=False,
    )

    transpose_k = functools.partial(
        pl.kernel,
        mesh=mesh,
        out_type=jax.ShapeDtypeStruct((1000000 * 128,), jnp.float32),
        scratch_types=[
            pltpu.VMEM((D_MODEL, 128), jnp.float32),
            pltpu.VMEM((128 * 128,), jnp.float32),
            pltpu.SemaphoreType.DMA,
        ],
        compiler_params=params,
    )(functools.partial(_transpose_body, nc=nc))

    gather_k = functools.partial(
        pl.kernel,
        mesh=mesh,
        out_type=jax.ShapeDtypeStruct((50 * 8 * 128 * 8 * 128,), jnp.float32),
        scratch_types=[
            pltpu.VMEM((256,), jnp.int32),
            pltpu.VMEM((256, 128), jnp.float32),
            pltpu.VMEM((128 * 128,), jnp.float32),
            pltpu.SemaphoreType.DMA,
        ],
        compiler_params=params,
    )(functools.partial(_gather_body, nc=nc))

    scr = transpose_k(tt)
    out1 = gather_k(scr.reshape(1000000, 128), idx_t)
    out5 = out1.reshape(50, 8, 128, 8, 128)
    return out5.transpose(2, 4, 0, 1, 3).reshape(b, s, D_MODEL)


# speedup vs baseline: 1.3773x; 1.3773x over previous
import functools

import jax
import jax.numpy as jnp
from jax import lax
from jax.experimental import pallas as pl
from jax.experimental.pallas import tpu as pltpu
from jax.experimental.pallas import tpu_sc as plsc

D_MODEL = 64
LANES = 16
SCALE = 8.0

SLAB_W = 128
N_FULL_SLABS = 7812
VOCAB_TAIL = 999936
SLABS_PER_W = 245
K1_PAIRS = 123

CHUNK = 256


def _iota16():
    return lax.iota(jnp.int32, LANES)


def _splat16(v):
    return jnp.zeros((LANES,), jnp.int32) + v


def _transpose_compute(slab_v, trows_v, nrows, iota):
    ivecs = [g * LANES + iota for g in range(SLAB_W // LANES)]
    ng = nrows // LANES
    for d in range(D_MODEL):
        dvec = _splat16(d)
        for g in range(ng):
            v = slab_v[d, pl.ds(g * LANES, LANES)]
            plsc.store_scatter(trows_v, [ivecs[g], dvec], v * SCALE)


def _transpose_body(tt_hbm, scr_hbm, slab0, slab1, trows0, trows1,
                    in0, in1, out0, out1, *, nc):
    wid = lax.axis_index("s") * nc + lax.axis_index("c")
    t0 = wid * SLABS_PER_W
    t1 = jnp.minimum(t0 + SLABS_PER_W, N_FULL_SLABS)
    iota = _iota16()
    slabs = (slab0, slab1)
    trows = (trows0, trows1)
    in_sems = (in0, in1)
    out_sems = (out0, out1)

    def in_copy(c, b):
        return pltpu.make_async_copy(
            tt_hbm.at[:, pl.ds(c * SLAB_W, SLAB_W)], slabs[b], in_sems[b]
        )

    def out_copy(c, b):
        return pltpu.make_async_copy(
            trows[b], scr_hbm.at[pl.ds(c * SLAB_W, SLAB_W), :], out_sems[b]
        )

    @pl.when(t0 < t1)
    def _prime():
        in_copy(t0, 0).start()

    def pair_body(g, carry):
        for b in range(2):
            c = t0 + g * 2 + b

            @pl.when(c < t1)
            def _step():
                in_copy(c, b).wait()

                @pl.when(c + 1 < t1)
                def _next():
                    in_copy(c + 1, 1 - b).start()

                @pl.when(c - t0 >= 2)
                def _drain():
                    out_copy(c - 2, b).wait()

                _transpose_compute(slabs[b], trows[b], SLAB_W, iota)
                out_copy(c, b).start()

        return carry

    lax.fori_loop(0, K1_PAIRS, pair_body, 0)

    out_copy(t1 - 2, 1).wait()
    out_copy(t1 - 1, 0).wait()

    @pl.when(wid == 31)
    def _tail():
        t_start = pl.multiple_of(wid * 0 + VOCAB_TAIL, 128)
        pltpu.make_async_copy(
            tt_hbm.at[:, pl.ds(t_start, 128)], slab0, in0
        ).start()
        pltpu.make_async_copy(
            tt_hbm.at[:, pl.ds(t_start, 128)], slab0, in0
        ).wait()
        _transpose_compute(slab0, trows0, 64, iota)
        pltpu.make_async_copy(
            trows0.at[pl.ds(0, 64), :],
            scr_hbm.at[pl.ds(VOCAB_TAIL, 64), :],
            out0,
        ).start()
        pltpu.make_async_copy(
            trows0.at[pl.ds(0, 64), :],
            scr_hbm.at[pl.ds(VOCAB_TAIL, 64), :],
            out0,
        ).wait()


def _gather_compute(rows_v, trows_v, iota):
    def bg_body(bg, carry):
        bvec = bg * LANES + iota
        boff = (bg // 8) * 1024 + (bg % 8) * LANES
        for d in range(D_MODEL):
            v = plsc.load_gather(rows_v, [bvec, _splat16(d)])
            off = (d // 8) * 2048 + (d % 8) * 128 + boff
            trows_v[pl.ds(off, LANES)] = v
        return carry

    lax.fori_loop(0, CHUNK // LANES, bg_body, 0, unroll=2)


def _gather_body(scr_hbm, idx_hbm, out_hbm, idx0, idx1, rows0, rows1,
                 trows0, trows1, isem0, isem1, rsem0, rsem1, osem0, osem1,
                 *, nc):
    wid = lax.axis_index("s") * nc + lax.axis_index("c")
    bh0 = wid * 4
    iota = _iota16()
    idxs = (idx0, idx1)
    rows = (rows0, rows1)
    trows = (trows0, trows1)
    isems = (isem0, isem1)
    rsems = (rsem0, rsem1)
    osems = (osem0, osem1)

    def idx_copy(t, b):
        s = t // 2
        b0 = (bh0 + (t % 2) * 2) * 128
        return pltpu.make_async_copy(
            idx_hbm.at[pl.ds(s * 16384 + b0, CHUNK)], idxs[b], isems[b]
        )

    def row_copy(b):
        return pltpu.make_async_copy(scr_hbm.at[idxs[b]], rows[b], rsems[b])

    def out_copies(t, b):
        s = t // 2
        obase = s * 1048576 + (bh0 + (t % 2) * 2) * 1024
        return [
            pltpu.make_async_copy(
                trows[b].at[pl.ds(dh * 2048, 2048)],
                out_hbm.at[pl.ds(obase + dh * 131072, 2048)],
                osems[b],
            )
            for dh in range(8)
        ]

    idx_copy(0, 0).start()
    idx_copy(0, 0).wait()
    row_copy(0).start()
    idx_copy(1, 1).start()

    def task_body(ti, carry):
        for b in range(2):
            t = ti * 2 + b

            @pl.when(t + 1 < 100)
            def _next_gather():
                idx_copy(t + 1, 1 - b).wait()
                row_copy(1 - b).start()

            row_copy(b).wait()

            @pl.when(t + 2 < 100)
            def _next_idx():
                idx_copy(t + 2, b).start()

            @pl.when(t >= 2)
            def _drain():
                for cp in out_copies(t - 2, b):
                    cp.wait()

            _gather_compute(rows[b], trows[b], iota)
            for cp in out_copies(t, b):
                cp.start()

        return carry

    lax.fori_loop(0, 50, task_body, 0)

    for t in (98, 99):
        for cp in out_copies(t, t % 2):
            cp.wait()


def kernel(x, table):
    b, s = x.shape
    n = b * s
    idx_t = jnp.transpose(x).reshape(n).astype(jnp.int32)
    tt = jnp.transpose(table)

    info = plsc.get_sparse_core_info()
    nc = info.num_cores
    mesh = plsc.VectorSubcoreMesh(core_axis_name="c", subcore_axis_name="s")
    params = pltpu.CompilerParams(
        use_tc_tiling_on_sc=True,
        disable_bounds_checks=True,
        needs_layout_passes=False,
    )

    transpose_k = functools.partial(
        pl.kernel,
        mesh=mesh,
        out_type=jax.ShapeDtypeStruct((1000000, 128), jnp.float32),
        scratch_types=[
            pltpu.VMEM((D_MODEL, SLAB_W), jnp.float32),
            pltpu.VMEM((D_MODEL, SLAB_W), jnp.float32),
            pltpu.VMEM((SLAB_W, 128), jnp.float32),
            pltpu.VMEM((SLAB_W, 128), jnp.float32),
            pltpu.SemaphoreType.DMA,
            pltpu.SemaphoreType.DMA,
            pltpu.SemaphoreType.DMA,
            pltpu.SemaphoreType.DMA,
        ],
        compiler_params=params,
    )(functools.partial(_transpose_body, nc=nc))

    gather_k = functools.partial(
        pl.kernel,
        mesh=mesh,
        out_type=jax.ShapeDtypeStruct((50 * 8 * 128 * 8 * 128,), jnp.float32),
        scratch_types=[
            pltpu.VMEM((CHUNK,), jnp.int32),
            pltpu.VMEM((CHUNK,), jnp.int32),
            pltpu.VMEM((CHUNK, 128), jnp.float32),
            pltpu.VMEM((CHUNK, 128), jnp.float32),
            pltpu.VMEM((128 * 128,), jnp.float32),
            pltpu.VMEM((128 * 128,), jnp.float32),
            pltpu.SemaphoreType.DMA,
            pltpu.SemaphoreType.DMA,
            pltpu.SemaphoreType.DMA,
            pltpu.SemaphoreType.DMA,
            pltpu.SemaphoreType.DMA,
            pltpu.SemaphoreType.DMA,
        ],
        compiler_params=params,
    )(functools.partial(_gather_body, nc=nc))

    scr = transpose_k(tt)
    out1 = gather_k(scr, idx_t)
    out5 = out1.reshape(50, 8, 128, 8, 128)
    return out5.transpose(2, 4, 0, 1, 3).reshape(b, s, D_MODEL)


# speedup vs baseline: 2.5162x; 1.8268x over previous
import functools

import jax
import jax.numpy as jnp
from jax import lax
from jax.experimental import pallas as pl
from jax.experimental.pallas import tpu as pltpu
from jax.experimental.pallas import tpu_sc as plsc

D_MODEL = 64
LANES = 16
SCALE = 8.0

SLAB_W = 128
N_FULL_SLABS = 7812
VOCAB_TAIL = 999936
SLABS_PER_W = 245
K1_PAIRS = 123

CHUNK = 256


def _iota16():
    return lax.iota(jnp.int32, LANES)


def _splat16(v):
    return jnp.zeros((LANES,), jnp.int32) + v


def _transpose_compute(slab_v, trows_v, nrows, iota):
    gvecs = [g * LANES + iota for g in range(D_MODEL // LANES)]

    @plsc.parallel_loop(0, nrows, unroll=4)
    def _il_loop(il):
        ilv = _splat16(il)
        for g in range(D_MODEL // LANES):
            v = plsc.load_gather(slab_v, [gvecs[g], ilv])
            trows_v[pl.ds(il * 128 + g * LANES, LANES)] = v * SCALE


def _transpose_body(tt_hbm, scr_hbm, slab0, slab1, trows0, trows1,
                    in0, in1, out0, out1, *, nc):
    wid = lax.axis_index("s") * nc + lax.axis_index("c")
    t0 = wid * SLABS_PER_W
    t1 = jnp.minimum(t0 + SLABS_PER_W, N_FULL_SLABS)
    iota = _iota16()
    slabs = (slab0, slab1)
    trows = (trows0, trows1)
    in_sems = (in0, in1)
    out_sems = (out0, out1)

    def in_copy(c, b):
        return pltpu.make_async_copy(
            tt_hbm.at[:, pl.ds(c * SLAB_W, SLAB_W)], slabs[b], in_sems[b]
        )

    def out_copy(c, b):
        return pltpu.make_async_copy(
            trows[b],
            scr_hbm.at[pl.ds(c * (SLAB_W * 128), SLAB_W * 128)],
            out_sems[b],
        )

    @pl.when(t0 < t1)
    def _prime():
        in_copy(t0, 0).start()

    def pair_body(g, carry):
        for b in range(2):
            c = t0 + g * 2 + b

            @pl.when(c < t1)
            def _step():
                in_copy(c, b).wait()

                @pl.when(c + 1 < t1)
                def _next():
                    in_copy(c + 1, 1 - b).start()

                @pl.when(c - t0 >= 2)
                def _drain():
                    out_copy(c - 2, b).wait()

                _transpose_compute(slabs[b], trows[b], SLAB_W, iota)
                out_copy(c, b).start()

        return carry

    lax.fori_loop(0, K1_PAIRS, pair_body, 0)

    out_copy(t1 - 2, 1).wait()
    out_copy(t1 - 1, 0).wait()

    @pl.when(wid == 31)
    def _tail():
        t_start = pl.multiple_of(wid * 0 + VOCAB_TAIL, 128)
        pltpu.make_async_copy(
            tt_hbm.at[:, pl.ds(t_start, 128)], slab0, in0
        ).start()
        pltpu.make_async_copy(
            tt_hbm.at[:, pl.ds(t_start, 128)], slab0, in0
        ).wait()
        _transpose_compute(slab0, trows0, 64, iota)
        pltpu.make_async_copy(
            trows0.at[pl.ds(0, 64 * 128)],
            scr_hbm.at[pl.ds(VOCAB_TAIL * 128, 64 * 128)],
            out0,
        ).start()
        pltpu.make_async_copy(
            trows0.at[pl.ds(0, 64 * 128)],
            scr_hbm.at[pl.ds(VOCAB_TAIL * 128, 64 * 128)],
            out0,
        ).wait()


def _gather_compute(rows_v, trows_v, iota):
    bvecs = [bg * LANES + iota for bg in range(CHUNK // LANES)]
    boffs = [(bg // 8) * 1024 + (bg % 8) * LANES for bg in range(CHUNK // LANES)]

    @plsc.parallel_loop(0, D_MODEL, unroll=2)
    def _d_loop(d):
        dvec = _splat16(d)
        dbase = (d // 8) * 2048 + (d % 8) * 128
        for bg in range(CHUNK // LANES):
            v = plsc.load_gather(rows_v, [bvecs[bg], dvec])
            trows_v[pl.ds(dbase + boffs[bg], LANES)] = v


def _gather_body(scr_hbm, idx_hbm, out_hbm, idx0, idx1, rows0, rows1,
                 trows0, trows1, isem0, isem1, rsem0, rsem1, osem0, osem1,
                 *, nc):
    wid = lax.axis_index("s") * nc + lax.axis_index("c")
    bh0 = wid * 4
    iota = _iota16()
    idxs = (idx0, idx1)
    rows = (rows0, rows1)
    trows = (trows0, trows1)
    isems = (isem0, isem1)
    rsems = (rsem0, rsem1)
    osems = (osem0, osem1)

    def idx_copy(t, b):
        s = t // 2
        b0 = (bh0 + (t % 2) * 2) * 128
        return pltpu.make_async_copy(
            idx_hbm.at[pl.ds(s * 16384 + b0, CHUNK)], idxs[b], isems[b]
        )

    def row_copy(b):
        return pltpu.make_async_copy(scr_hbm.at[idxs[b]], rows[b], rsems[b])

    def out_copies(t, b):
        s = t // 2
        obase = s * 1048576 + (bh0 + (t % 2) * 2) * 1024
        return [
            pltpu.make_async_copy(
                trows[b].at[pl.ds(dh * 2048, 2048)],
                out_hbm.at[pl.ds(obase + dh * 131072, 2048)],
                osems[b],
            )
            for dh in range(8)
        ]

    idx_copy(0, 0).start()
    idx_copy(0, 0).wait()
    row_copy(0).start()
    idx_copy(1, 1).start()

    def task_body(ti, carry):
        for b in range(2):
            t = ti * 2 + b

            @pl.when(t + 1 < 100)
            def _next_gather():
                idx_copy(t + 1, 1 - b).wait()
                row_copy(1 - b).start()

            row_copy(b).wait()

            @pl.when(t + 2 < 100)
            def _next_idx():
                idx_copy(t + 2, b).start()

            @pl.when(t >= 2)
            def _drain():
                for cp in out_copies(t - 2, b):
                    cp.wait()

            _gather_compute(rows[b], trows[b], iota)
            for cp in out_copies(t, b):
                cp.start()

        return carry

    lax.fori_loop(0, 50, task_body, 0)

    for t in (98, 99):
        for cp in out_copies(t, t % 2):
            cp.wait()


def kernel(x, table):
    b, s = x.shape
    n = b * s
    idx_t = jnp.transpose(x).reshape(n).astype(jnp.int32)
    tt = jnp.transpose(table)

    info = plsc.get_sparse_core_info()
    nc = info.num_cores
    mesh = plsc.VectorSubcoreMesh(core_axis_name="c", subcore_axis_name="s")
    params = pltpu.CompilerParams(
        use_tc_tiling_on_sc=True,
        disable_bounds_checks=True,
        needs_layout_passes=False,
    )

    transpose_k = functools.partial(
        pl.kernel,
        mesh=mesh,
        out_type=jax.ShapeDtypeStruct((1000000 * 128,), jnp.float32),
        scratch_types=[
            pltpu.VMEM((D_MODEL, SLAB_W), jnp.float32),
            pltpu.VMEM((D_MODEL, SLAB_W), jnp.float32),
            pltpu.VMEM((SLAB_W * 128,), jnp.float32),
            pltpu.VMEM((SLAB_W * 128,), jnp.float32),
            pltpu.SemaphoreType.DMA,
            pltpu.SemaphoreType.DMA,
            pltpu.SemaphoreType.DMA,
            pltpu.SemaphoreType.DMA,
        ],
        compiler_params=params,
    )(functools.partial(_transpose_body, nc=nc))

    gather_k = functools.partial(
        pl.kernel,
        mesh=mesh,
        out_type=jax.ShapeDtypeStruct((50 * 8 * 128 * 8 * 128,), jnp.float32),
        scratch_types=[
            pltpu.VMEM((CHUNK,), jnp.int32),
            pltpu.VMEM((CHUNK,), jnp.int32),
            pltpu.VMEM((CHUNK, 128), jnp.float32),
            pltpu.VMEM((CHUNK, 128), jnp.float32),
            pltpu.VMEM((128 * 128,), jnp.float32),
            pltpu.VMEM((128 * 128,), jnp.float32),
            pltpu.SemaphoreType.DMA,
            pltpu.SemaphoreType.DMA,
            pltpu.SemaphoreType.DMA,
            pltpu.SemaphoreType.DMA,
            pltpu.SemaphoreType.DMA,
            pltpu.SemaphoreType.DMA,
        ],
        compiler_params=params,
    )(functools.partial(_gather_body, nc=nc))

    scr = transpose_k(tt)
    out1 = gather_k(scr.reshape(1000000, 128), idx_t)
    out5 = out1.reshape(50, 8, 128, 8, 128)
    return out5.transpose(2, 4, 0, 1, 3).reshape(b, s, D_MODEL)


# speedup vs baseline: 2.5220x; 1.0023x over previous
import functools

import jax
import jax.numpy as jnp
from jax import lax
from jax.experimental import pallas as pl
from jax.experimental.pallas import tpu as pltpu
from jax.experimental.pallas import tpu_sc as plsc

D_MODEL = 64
LANES = 16
SCALE = 8.0

SLAB_W = 128
N_FULL_SLABS = 7812
VOCAB_TAIL = 999936
SLABS_PER_W = 245
K1_PAIRS = 123

CHUNK = 256


def _iota16():
    return lax.iota(jnp.int32, LANES)


def _splat16(v):
    return jnp.zeros((LANES,), jnp.int32) + v


def _transpose_compute(slab_v, trows_v, nrows, iota):
    gvecs = [g * LANES + iota for g in range(D_MODEL // LANES)]

    @plsc.parallel_loop(0, nrows, unroll=4)
    def _il_loop(il):
        ilv = _splat16(il)
        for g in range(D_MODEL // LANES):
            v = plsc.load_gather(slab_v, [gvecs[g], ilv])
            trows_v[pl.ds(il * D_MODEL + g * LANES, LANES)] = v * SCALE


def _transpose_body(tt_hbm, scr_hbm, slab0, slab1, trows0, trows1,
                    in0, in1, out0, out1, *, nc):
    wid = lax.axis_index("s") * nc + lax.axis_index("c")
    t0 = wid * SLABS_PER_W
    t1 = jnp.minimum(t0 + SLABS_PER_W, N_FULL_SLABS)
    iota = _iota16()
    slabs = (slab0, slab1)
    trows = (trows0, trows1)
    in_sems = (in0, in1)
    out_sems = (out0, out1)

    def in_copy(c, b):
        return pltpu.make_async_copy(
            tt_hbm.at[:, pl.ds(c * SLAB_W, SLAB_W)], slabs[b], in_sems[b]
        )

    def out_copy(c, b):
        return pltpu.make_async_copy(
            trows[b],
            scr_hbm.at[pl.ds(c * (SLAB_W * D_MODEL), SLAB_W * D_MODEL)],
            out_sems[b],
        )

    @pl.when(t0 < t1)
    def _prime():
        in_copy(t0, 0).start()

    def pair_body(g, carry):
        for b in range(2):
            c = t0 + g * 2 + b

            @pl.when(c < t1)
            def _step():
                in_copy(c, b).wait()

                @pl.when(c + 1 < t1)
                def _next():
                    in_copy(c + 1, 1 - b).start()

                @pl.when(c - t0 >= 2)
                def _drain():
                    out_copy(c - 2, b).wait()

                _transpose_compute(slabs[b], trows[b], SLAB_W, iota)
                out_copy(c, b).start()

        return carry

    lax.fori_loop(0, K1_PAIRS, pair_body, 0)

    out_copy(t1 - 2, 1).wait()
    out_copy(t1 - 1, 0).wait()

    @pl.when(wid == 31)
    def _tail():
        t_start = pl.multiple_of(wid * 0 + VOCAB_TAIL, 128)
        pltpu.make_async_copy(
            tt_hbm.at[:, pl.ds(t_start, 128)], slab0, in0
        ).start()
        pltpu.make_async_copy(
            tt_hbm.at[:, pl.ds(t_start, 128)], slab0, in0
        ).wait()
        _transpose_compute(slab0, trows0, 64, iota)
        pltpu.make_async_copy(
            trows0.at[pl.ds(0, 64 * D_MODEL)],
            scr_hbm.at[pl.ds(VOCAB_TAIL * D_MODEL, 64 * D_MODEL)],
            out0,
        ).start()
        pltpu.make_async_copy(
            trows0.at[pl.ds(0, 64 * D_MODEL)],
            scr_hbm.at[pl.ds(VOCAB_TAIL * D_MODEL, 64 * D_MODEL)],
            out0,
        ).wait()


def _gather_compute(rows_v, trows_v, iota):
    bvecs = [bg * LANES + iota for bg in range(CHUNK // LANES)]
    boffs = [(bg // 8) * 1024 + (bg % 8) * LANES for bg in range(CHUNK // LANES)]

    @plsc.parallel_loop(0, D_MODEL, unroll=2)
    def _d_loop(d):
        dvec = _splat16(d)
        dbase = (d // 8) * 2048 + (d % 8) * 128
        for bg in range(CHUNK // LANES):
            v = plsc.load_gather(rows_v, [bvecs[bg], dvec])
            trows_v[pl.ds(dbase + boffs[bg], LANES)] = v


def _gather_body(scr_hbm, idx_hbm, out_hbm, idx0, idx1, rows0, rows1,
                 trows0, trows1, isem0, isem1, rsem0, rsem1, osem0, osem1,
                 *, nc):
    wid = lax.axis_index("s") * nc + lax.axis_index("c")
    bh0 = wid * 4
    iota = _iota16()
    idxs = (idx0, idx1)
    rows = (rows0, rows1)
    trows = (trows0, trows1)
    isems = (isem0, isem1)
    rsems = (rsem0, rsem1)
    osems = (osem0, osem1)

    def idx_copy(t, b):
        s = t // 2
        b0 = (bh0 + (t % 2) * 2) * 128
        return pltpu.make_async_copy(
            idx_hbm.at[pl.ds(s * 16384 + b0, CHUNK)], idxs[b], isems[b]
        )

    def row_copy(b):
        return pltpu.make_async_copy(scr_hbm.at[idxs[b]], rows[b], rsems[b])

    def out_copies(t, b):
        s = t // 2
        obase = s * 1048576 + (bh0 + (t % 2) * 2) * 1024
        return [
            pltpu.make_async_copy(
                trows[b].at[pl.ds(dh * 2048, 2048)],
                out_hbm.at[pl.ds(obase + dh * 131072, 2048)],
                osems[b],
            )
            for dh in range(8)
        ]

    idx_copy(0, 0).start()
    idx_copy(0, 0).wait()
    row_copy(0).start()
    idx_copy(1, 1).start()

    def task_body(ti, carry):
        for b in range(2):
            t = ti * 2 + b

            @pl.when(t + 1 < 100)
            def _next_gather():
                idx_copy(t + 1, 1 - b).wait()
                row_copy(1 - b).start()

            row_copy(b).wait()

            @pl.when(t + 2 < 100)
            def _next_idx():
                idx_copy(t + 2, b).start()

            @pl.when(t >= 2)
            def _drain():
                for cp in out_copies(t - 2, b):
                    cp.wait()

            _gather_compute(rows[b], trows[b], iota)
            for cp in out_copies(t, b):
                cp.start()

        return carry

    lax.fori_loop(0, 50, task_body, 0)

    for t in (98, 99):
        for cp in out_copies(t, t % 2):
            cp.wait()


def kernel(x, table):
    b, s = x.shape
    n = b * s
    idx_t = jnp.transpose(x).reshape(n).astype(jnp.int32)
    tt = jnp.transpose(table)

    info = plsc.get_sparse_core_info()
    nc = info.num_cores
    mesh = plsc.VectorSubcoreMesh(core_axis_name="c", subcore_axis_name="s")
    params = pltpu.CompilerParams(
        use_tc_tiling_on_sc=True,
        disable_bounds_checks=True,
        needs_layout_passes=False,
    )
    params_linear = pltpu.CompilerParams(
        use_tc_tiling_on_sc=False,
        needs_layout_passes=False,
    )

    transpose_k = functools.partial(
        pl.kernel,
        mesh=mesh,
        out_type=jax.ShapeDtypeStruct((1000000 * D_MODEL,), jnp.float32),
        scratch_types=[
            pltpu.VMEM((D_MODEL, SLAB_W), jnp.float32),
            pltpu.VMEM((D_MODEL, SLAB_W), jnp.float32),
            pltpu.VMEM((SLAB_W * D_MODEL,), jnp.float32),
            pltpu.VMEM((SLAB_W * D_MODEL,), jnp.float32),
            pltpu.SemaphoreType.DMA,
            pltpu.SemaphoreType.DMA,
            pltpu.SemaphoreType.DMA,
            pltpu.SemaphoreType.DMA,
        ],
        compiler_params=params,
    )(functools.partial(_transpose_body, nc=nc))

    gather_k = functools.partial(
        pl.kernel,
        mesh=mesh,
        out_type=jax.ShapeDtypeStruct((50 * 8 * 128 * 8 * 128,), jnp.float32),
        scratch_types=[
            pltpu.VMEM((CHUNK,), jnp.int32),
            pltpu.VMEM((CHUNK,), jnp.int32),
            pltpu.VMEM((CHUNK, D_MODEL), jnp.float32),
            pltpu.VMEM((CHUNK, D_MODEL), jnp.float32),
            pltpu.VMEM((128 * 128,), jnp.float32),
            pltpu.VMEM((128 * 128,), jnp.float32),
            pltpu.SemaphoreType.DMA,
            pltpu.SemaphoreType.DMA,
            pltpu.SemaphoreType.DMA,
            pltpu.SemaphoreType.DMA,
            pltpu.SemaphoreType.DMA,
            pltpu.SemaphoreType.DMA,
        ],
        compiler_params=params_linear,
    )(functools.partial(_gather_body, nc=nc))

    scr = transpose_k(tt)
    out1 = gather_k(scr.reshape(1000000, D_MODEL), idx_t)
    out5 = out1.reshape(50, 8, 128, 8, 128)
    return out5.transpose(2, 4, 0, 1, 3).reshape(b, s, D_MODEL)
